# baseline (speedup 1.0000x reference)
import jax
import jax.numpy as jnp
from jax.experimental import pallas as pl
from jax.experimental.pallas import tpu as pltpu

_D = 128
_NUM_STAGES = 4
_BLOCK = 16384
_SUB = 1024


def _msrc_kernel(c_ref, x_ref, r_ref, o_ref):
    R = r_ref[...]
    sign_mask = jnp.int32(-2147483648)
    pos_bits = [
        jax.lax.bitcast_convert_type(c_ref[s, 1], jnp.int32)
        for s in range(_NUM_STAGES)
    ]

    def body(t, carry):
        base = t * _SUB
        x = x_ref[pl.ds(base, _SUB), :]
        ssq = jnp.sum(x * x, axis=1, keepdims=True)
        inv = jax.lax.rsqrt(ssq)
        nrm = ssq * inv
        xn = x * inv
        xr = jax.lax.dot_general(
            xn, R, (((1,), (0,)), ((), ())),
            preferred_element_type=jnp.float32)
        residual = xr
        for s in range(_NUM_STAGES):
            r_bits = jax.lax.bitcast_convert_type(residual, jnp.int32)
            recon = jax.lax.bitcast_convert_type(
                (r_bits & sign_mask) ^ pos_bits[s], jnp.float32)
            residual = residual - recon
        combined = xr - residual
        out = jax.lax.dot_general(
            combined, R, (((1,), (1,)), ((), ())),
            preferred_element_type=jnp.float32)
        o_ref[pl.ds(base, _SUB), :] = out * nrm
        return carry

    jax.lax.fori_loop(0, _BLOCK // _SUB, body, 0)


def kernel(x, R, centroids):
    n, d = x.shape
    grid = (n // _BLOCK,)
    return pl.pallas_call(
        _msrc_kernel,
        grid=grid,
        in_specs=[
            pl.BlockSpec(memory_space=pltpu.SMEM),
            pl.BlockSpec((_BLOCK, d), lambda i: (i, 0)),
            pl.BlockSpec((d, d), lambda i: (0, 0)),
        ],
        out_specs=pl.BlockSpec((_BLOCK, d), lambda i: (i, 0)),
        out_shape=jax.ShapeDtypeStruct((n, d), jnp.float32),
        compiler_params=pltpu.CompilerParams(
            dimension_semantics=("arbitrary",)),
    )(centroids, x, R)


# speedup vs baseline: 1.4515x; 1.4515x over previous
import jax
import jax.numpy as jnp
from jax.experimental import pallas as pl
from jax.experimental.pallas import tpu as pltpu

_D = 128
_NUM_STAGES = 4
_BLOCK = 16384
_SUB = 1024


def _msrc_kernel(c_ref, x_ref, r_ref, o_ref):
    R = r_ref[...]
    sign_mask = jnp.int32(-2147483648)
    pos_bits = [
        jax.lax.bitcast_convert_type(c_ref[s, 1], jnp.int32)
        for s in range(_NUM_STAGES)
    ]

    def body(base):
        x = x_ref[pl.ds(base, _SUB), :]
        ssq = jnp.sum(x * x, axis=1, keepdims=True)
        inv = jax.lax.rsqrt(ssq)
        nrm = ssq * inv
        xn = x * inv
        xr = jax.lax.dot_general(
            xn, R, (((1,), (0,)), ((), ())),
            preferred_element_type=jnp.float32)
        residual = xr
        for s in range(_NUM_STAGES):
            r_bits = jax.lax.bitcast_convert_type(residual, jnp.int32)
            recon = jax.lax.bitcast_convert_type(
                (r_bits & sign_mask) ^ pos_bits[s], jnp.float32)
            residual = residual - recon
        combined = xr - residual
        out = jax.lax.dot_general(
            combined, R, (((1,), (1,)), ((), ())),
            preferred_element_type=jnp.float32)
        o_ref[pl.ds(base, _SUB), :] = out * nrm

    for t in range(_BLOCK // _SUB):
        body(t * _SUB)


def kernel(x, R, centroids):
    n, d = x.shape
    grid = (n // _BLOCK,)
    return pl.pallas_call(
        _msrc_kernel,
        grid=grid,
        in_specs=[
            pl.BlockSpec(memory_space=pltpu.SMEM),
            pl.BlockSpec((_BLOCK, d), lambda i: (i, 0)),
            pl.BlockSpec((d, d), lambda i: (0, 0)),
        ],
        out_specs=pl.BlockSpec((_BLOCK, d), lambda i: (i, 0)),
        out_shape=jax.ShapeDtypeStruct((n, d), jnp.float32),
        compiler_params=pltpu.CompilerParams(
            dimension_semantics=("arbitrary",)),
    )(centroids, x, R)


# speedup vs baseline: 1.4536x; 1.0015x over previous
import jax
import jax.numpy as jnp
from jax.experimental import pallas as pl
from jax.experimental.pallas import tpu as pltpu

_D = 128
_NUM_STAGES = 4
_BLOCK = 16384


def _msrc_kernel(c_ref, x_ref, r_ref, o_ref):
    x = x_ref[...]
    R = r_ref[...]
    ssq = jnp.sum(x * x, axis=1, keepdims=True)
    inv = jax.lax.rsqrt(ssq)
    nrm = ssq * inv
    xn = x * inv
    xr = jax.lax.dot_general(
        xn, R, (((1,), (0,)), ((), ())),
        preferred_element_type=jnp.float32)
    sign_mask = jnp.int32(-2147483648)
    residual = xr
    for s in range(_NUM_STAGES):
        pos_bits = jax.lax.bitcast_convert_type(c_ref[s, 1], jnp.int32)
        r_bits = jax.lax.bitcast_convert_type(residual, jnp.int32)
        recon = jax.lax.bitcast_convert_type(
            (r_bits & sign_mask) ^ pos_bits, jnp.float32)
        residual = residual - recon
    combined = (xr - residual) * nrm
    o_ref[...] = jax.lax.dot_general(
        combined, R, (((1,), (1,)), ((), ())), preferred_element_type=jnp.float32)


def kernel(x, R, centroids):
    n, d = x.shape
    grid = (n // _BLOCK,)
    return pl.pallas_call(
        _msrc_kernel,
        grid=grid,
        in_specs=[
            pl.BlockSpec(memory_space=pltpu.SMEM),
            pl.BlockSpec((_BLOCK, d), lambda i: (i, 0)),
            pl.BlockSpec((d, d), lambda i: (0, 0)),
        ],
        out_specs=pl.BlockSpec((_BLOCK, d), lambda i: (i, 0)),
        out_shape=jax.ShapeDtypeStruct((n, d), jnp.float32),
        compiler_params=pltpu.CompilerParams(
            dimension_semantics=("arbitrary",)),
    )(centroids, x, R)


# speedup vs baseline: 1.4663x; 1.0087x over previous
import jax
import jax.numpy as jnp
from jax.experimental import pallas as pl
from jax.experimental.pallas import tpu as pltpu

_D = 128
_NUM_STAGES = 4
_BLOCK = 16384


def _msrc_kernel(c_ref, x_ref, r_ref, o_ref):
    x = x_ref[...]
    R = r_ref[...]
    ssq = jnp.sum(x * x, axis=1, keepdims=True)
    inv = jax.lax.rsqrt(ssq)
    nrm = ssq * inv
    xn = x * inv
    xr = jax.lax.dot_general(
        xn, R, (((1,), (0,)), ((), ())),
        preferred_element_type=jnp.float32)
    sign_mask = jnp.int32(-2147483648)
    residual = xr
    for s in range(_NUM_STAGES):
        pos_bits = jax.lax.bitcast_convert_type(c_ref[s, 1], jnp.int32)
        r_bits = jax.lax.bitcast_convert_type(residual, jnp.int32)
        recon = jax.lax.bitcast_convert_type(
            (r_bits & sign_mask) ^ pos_bits, jnp.float32)
        residual = residual - recon
    combined = xr - residual
    out = jax.lax.dot_general(
        combined, R, (((1,), (1,)), ((), ())), preferred_element_type=jnp.float32)
    o_ref[...] = out * nrm


def kernel(x, R, centroids):
    n, d = x.shape
    grid = (n // _BLOCK,)
    return pl.pallas_call(
        _msrc_kernel,
        grid=grid,
        in_specs=[
            pl.BlockSpec(memory_space=pltpu.SMEM),
            pl.BlockSpec((_BLOCK, d), lambda i: (i, 0)),
            pl.BlockSpec((d, d), lambda i: (0, 0)),
        ],
        out_specs=pl.BlockSpec((_BLOCK, d), lambda i: (i, 0)),
        out_shape=jax.ShapeDtypeStruct((n, d), jnp.float32),
        compiler_params=pltpu.CompilerParams(
            dimension_semantics=("arbitrary",)),
    )(centroids, x, R)
